# baseline (speedup 1.0000x reference)
import functools

import jax
import jax.numpy as jnp
from jax import lax
from jax.experimental import pallas as pl
from jax.experimental.pallas import tpu as pltpu
from jax.experimental.pallas import tpu_sc as plsc

CH = 128
NBUF = 4


def _pq_body(x8_ref, wsb_ref, wrb_ref, p_ref, q_ref):
    x8 = x8_ref[...]
    p_ref[...] = jnp.dot(x8, wsb_ref[...], preferred_element_type=jnp.float32)
    q_ref[...] = jnp.dot(x8, wrb_ref[...], preferred_element_type=jnp.float32)


def _out_body(ea_ref, g_ref, wet_ref, bcol_ref, o_ref):
    acc = jnp.dot(wet_ref[...], ea_ref[...], preferred_element_type=jnp.float32)
    o_ref[...] = jnp.maximum(acc + g_ref[...] + bcol_ref[...], 0.0)


def _make_sc_gather_add(n_nodes, n_edges, d_out, n_workers):
    mesh = plsc.VectorSubcoreMesh(core_axis_name="c", subcore_axis_name="s")
    nchunks = n_edges // CH
    base_cnt = nchunks // n_workers
    rem = nchunks % n_workers
    cnt_max = base_cnt + (1 if rem else 0)
    tmax = -(-cnt_max // NBUF) * NBUF
    sub_tiles = 16 // 8

    scratch_types = [
        pltpu.VMEM((cnt_max, 2, CH), jnp.int32),
        pltpu.VMEM((NBUF, CH, d_out), jnp.float32),
        pltpu.VMEM((NBUF, CH, d_out), jnp.float32),
        pltpu.VMEM((NBUF, d_out * CH), jnp.float32),
    ] + [pltpu.SemaphoreType.DMA] * (2 * NBUF + 1)

    @functools.partial(
        pl.kernel,
        out_type=jax.ShapeDtypeStruct((sub_tiles, nchunks, 8 * 128), jnp.float32),
        mesh=mesh,
        scratch_types=scratch_types,
        compiler_params=pltpu.CompilerParams(
            use_tc_tiling_on_sc=False, needs_layout_passes=False),
    )
    def sc_kernel(p_hbm, q_hbm, ei3_hbm, gt_hbm,
                  slab, prow, qrow, obuf, *sems):
        semg = sems[:NBUF]
        semo = sems[NBUF:2 * NBUF]
        sem_stage = sems[2 * NBUF]
        wid = lax.axis_index("s") * 2 + lax.axis_index("c")
        cnt = base_cnt + (wid < rem).astype(jnp.int32)

        def chunk_id(t):
            return wid + n_workers * t

        def stage_fire(k, carry):
            pltpu.make_async_copy(
                ei3_hbm.at[chunk_id(k)], slab.at[k], sem_stage).start()
            return carry

        def stage_wait(k, carry):
            pltpu.make_async_copy(
                ei3_hbm.at[chunk_id(k)], slab.at[k], sem_stage).wait()
            return carry

        lax.fori_loop(0, cnt, stage_fire, 0)
        lax.fori_loop(0, cnt, stage_wait, 0)

        def fire_gathers(t, b):
            pltpu.make_async_copy(
                p_hbm.at[slab.at[t, 0]], prow.at[b], semg[b]).start()
            pltpu.make_async_copy(
                q_hbm.at[slab.at[t, 1]], qrow.at[b], semg[b]).start()

        def wait_gathers(t, b):
            pltpu.make_async_copy(
                p_hbm.at[slab.at[t, 0]], prow.at[b], semg[b]).wait()
            pltpu.make_async_copy(
                q_hbm.at[slab.at[t, 1]], qrow.at[b], semg[b]).wait()

        def out_copies(t, b):
            c = chunk_id(t)
            return [
                pltpu.make_async_copy(
                    obuf.at[b, pl.ds(a * 8 * 128, 8 * 128)],
                    gt_hbm.at[a, c],
                    semo[b],
                )
                for a in range(sub_tiles)
            ]

        for b in range(2):
            fire_gathers(b, b)

        ivec = lax.broadcasted_iota(jnp.int32, (16,), 0) * CH

        def step(go, carry):
            for b in range(NBUF):
                t = go * NBUF + b

                @pl.when(t < cnt)
                def _():
                    wait_gathers(t, b)

                b2 = (b + 2) % NBUF

                @pl.when(t + 2 < cnt)
                def _():
                    fire_gathers(t + 2, b2)

                @pl.when(jnp.logical_and(t >= NBUF, t < cnt))
                def _():
                    for cp in out_copies(t - NBUF, b):
                        cp.wait()

                @pl.when(t < cnt)
                def _():
                    pb = prow.at[b]
                    qb = qrow.at[b]
                    ob = obuf.at[b]

                    @plsc.parallel_loop(0, CH, step=1, unroll=8)
                    def _(i):
                        row = pb[i, :] + qb[i, :]
                        plsc.store_scatter(ob, [ivec + i], row)

                    for cp in out_copies(t, b):
                        cp.start()
            return carry

        lax.fori_loop(0, tmax // NBUF, step, 0)
        for b in range(NBUF):
            t_last = cnt - NBUF + ((b - cnt) % NBUF)

            @pl.when(t_last >= 0)
            def _():
                for cp in out_copies(t_last, b):
                    cp.wait()

    return sc_kernel


def kernel(x, edge_index, edge_attr, W, b):
    n_nodes, d_in = x.shape
    n_edges, d_edge = edge_attr.shape
    d_out = W.shape[1]

    w_s = W[:d_in]
    w_r = W[d_in:2 * d_in]
    w_e = W[2 * d_in:]
    ei3 = edge_index.reshape(n_edges // CH, CH, 2).transpose(0, 2, 1)
    ea_t = edge_attr.T
    wet = w_e.T
    bcol = b.reshape(d_out, 1)

    pack = 128 // d_out
    x8 = x.reshape(n_nodes // pack, pack * d_in)
    eye = jnp.eye(pack, dtype=jnp.float32)
    wsb = jnp.kron(eye, w_s)
    wrb = jnp.kron(eye, w_r)
    p128, q128 = pl.pallas_call(
        _pq_body,
        out_shape=(
            jax.ShapeDtypeStruct((n_nodes // pack, 128), jnp.float32),
            jax.ShapeDtypeStruct((n_nodes // pack, 128), jnp.float32),
        ),
    )(x8, wsb, wrb)
    p = p128.reshape(n_nodes, d_out)
    q = q128.reshape(n_nodes, d_out)

    info = plsc.get_sparse_core_info()
    n_workers = info.num_cores * info.num_subcores
    assert n_edges % CH == 0
    g4 = _make_sc_gather_add(n_nodes, n_edges, d_out, n_workers)(p, q, ei3)
    gt = (g4.reshape(2, n_edges // CH, 8, 128)
          .transpose(0, 2, 1, 3).reshape(d_out, n_edges))

    blk = 32000
    grid = n_edges // blk
    out_t = pl.pallas_call(
        _out_body,
        grid=(grid,),
        in_specs=[
            pl.BlockSpec((d_edge, blk), lambda i: (0, i)),
            pl.BlockSpec((d_out, blk), lambda i: (0, i)),
            pl.BlockSpec((d_out, d_edge), lambda i: (0, 0)),
            pl.BlockSpec((d_out, 1), lambda i: (0, 0)),
        ],
        out_specs=pl.BlockSpec((d_out, blk), lambda i: (0, i)),
        out_shape=jax.ShapeDtypeStruct((d_out, n_edges), jnp.float32),
    )(ea_t, gt, wet, bcol)
    return out_t.T


# speedup vs baseline: 1.4128x; 1.4128x over previous
import functools

import jax
import jax.numpy as jnp
from jax import lax
from jax.experimental import pallas as pl
from jax.experimental.pallas import tpu as pltpu
from jax.experimental.pallas import tpu_sc as plsc

CH = 128
NBUF = 6


def _pq_body(x8_ref, wsb_ref, wrb_ref, p_ref, q_ref):
    x8 = x8_ref[...]
    p_ref[...] = jnp.dot(x8, wsb_ref[...], preferred_element_type=jnp.float32)
    q_ref[...] = jnp.dot(x8, wrb_ref[...], preferred_element_type=jnp.float32)


def _out_body(ea_ref, g_ref, wet_ref, bcol_ref, o_ref):
    acc = jnp.dot(wet_ref[...], ea_ref[...], preferred_element_type=jnp.float32)
    o_ref[...] = jnp.maximum(acc + g_ref[...] + bcol_ref[...], 0.0)


def _make_sc_gather_add(n_nodes, n_edges, d_out, n_workers):
    mesh = plsc.VectorSubcoreMesh(core_axis_name="c", subcore_axis_name="s")
    nchunks = n_edges // CH
    base_cnt = nchunks // n_workers
    rem = nchunks % n_workers
    cnt_max = base_cnt + (1 if rem else 0)
    sub_tiles = 16 // 8
    assert base_cnt % NBUF == 0
    DEPTH = 4

    scratch_types = [
        pltpu.VMEM((cnt_max, 2, CH), jnp.int32),
        pltpu.VMEM((NBUF, CH, d_out), jnp.float32),
        pltpu.VMEM((NBUF, CH, d_out), jnp.float32),
        pltpu.VMEM((NBUF, d_out, CH), jnp.float32),
    ] + [pltpu.SemaphoreType.DMA] * (2 * NBUF + 1)

    @functools.partial(
        pl.kernel,
        out_type=jax.ShapeDtypeStruct((sub_tiles, nchunks, 8, 128), jnp.float32),
        mesh=mesh,
        scratch_types=scratch_types,
        compiler_params=pltpu.CompilerParams(
            use_tc_tiling_on_sc=False, needs_layout_passes=False),
    )
    def sc_kernel(p_hbm, q_hbm, ei3_hbm, gt_hbm,
                  slab, prow, qrow, obuf, *sems):
        semg = sems[:NBUF]
        semo = sems[NBUF:2 * NBUF]
        sem_stage = sems[2 * NBUF]
        wid = lax.axis_index("s") * 2 + lax.axis_index("c")
        cnt = base_cnt + (wid < rem).astype(jnp.int32)

        def chunk_id(t):
            return wid + n_workers * t

        def stage_fire(k, carry):
            pltpu.make_async_copy(
                ei3_hbm.at[chunk_id(k)], slab.at[k], sem_stage).start()
            return carry

        def stage_wait(k, carry):
            pltpu.make_async_copy(
                ei3_hbm.at[chunk_id(k)], slab.at[k], sem_stage).wait()
            return carry

        lax.fori_loop(0, cnt, stage_fire, 0)
        lax.fori_loop(0, cnt, stage_wait, 0)

        def fire_gathers(t, b):
            pltpu.make_async_copy(
                p_hbm.at[slab.at[t, 0]], prow.at[b], semg[b]).start()
            pltpu.make_async_copy(
                q_hbm.at[slab.at[t, 1]], qrow.at[b], semg[b]).start()

        def wait_gathers(t, b):
            pltpu.make_async_copy(
                p_hbm.at[slab.at[t, 0]], prow.at[b], semg[b]).wait()
            pltpu.make_async_copy(
                q_hbm.at[slab.at[t, 1]], qrow.at[b], semg[b]).wait()

        def out_copies(t, b):
            c = chunk_id(t)
            return [
                pltpu.make_async_copy(
                    obuf.at[b, pl.ds(a * 8, 8), :],
                    gt_hbm.at[a, c],
                    semo[b],
                )
                for a in range(sub_tiles)
            ]

        for b in range(DEPTH):
            fire_gathers(b, b)

        ivec = lax.broadcasted_iota(jnp.int32, (16,), 0)
        zvec = jnp.zeros((16,), jnp.int32)

        def compute_chunk(b):
            pb = prow.at[b]
            qb = qrow.at[b]
            ob = obuf.at[b]

            @plsc.parallel_loop(0, CH, step=1, unroll=8)
            def _(i):
                row = pb[i, :] + qb[i, :]
                plsc.store_scatter(ob, [ivec, zvec + i], row)

        def step(go, carry):
            for b in range(NBUF):
                t = go * NBUF + b
                wait_gathers(t, b)
                bd = (b + DEPTH) % NBUF

                @pl.when(t + DEPTH < cnt)
                def _():
                    fire_gathers(t + DEPTH, bd)

                @pl.when(go > 0)
                def _():
                    for cp in out_copies(t - NBUF, b):
                        cp.wait()

                compute_chunk(b)
                for cp in out_copies(t, b):
                    cp.start()
            return carry

        lax.fori_loop(0, base_cnt // NBUF, step, 0)

        tb = base_cnt % NBUF

        @pl.when(wid < rem)
        def _():
            wait_gathers(base_cnt, tb)
            for cp in out_copies(base_cnt - NBUF, tb):
                cp.wait()
            compute_chunk(tb)
            for cp in out_copies(base_cnt, tb):
                cp.start()

        for b in range(NBUF):
            for cp in out_copies(base_cnt - NBUF + b, b):
                cp.wait()

    return sc_kernel


def kernel(x, edge_index, edge_attr, W, b):
    n_nodes, d_in = x.shape
    n_edges, d_edge = edge_attr.shape
    d_out = W.shape[1]

    w_s = W[:d_in]
    w_r = W[d_in:2 * d_in]
    w_e = W[2 * d_in:]
    ei3 = edge_index.reshape(n_edges // CH, CH, 2).transpose(0, 2, 1)
    ea_t = edge_attr.T
    wet = w_e.T
    bcol = b.reshape(d_out, 1)

    pack = 128 // d_out
    x8 = x.reshape(n_nodes // pack, pack * d_in)
    eye = jnp.eye(pack, dtype=jnp.float32)
    wsb = jnp.kron(eye, w_s)
    wrb = jnp.kron(eye, w_r)
    p128, q128 = pl.pallas_call(
        _pq_body,
        out_shape=(
            jax.ShapeDtypeStruct((n_nodes // pack, 128), jnp.float32),
            jax.ShapeDtypeStruct((n_nodes // pack, 128), jnp.float32),
        ),
    )(x8, wsb, wrb)
    p = p128.reshape(n_nodes, d_out)
    q = q128.reshape(n_nodes, d_out)

    info = plsc.get_sparse_core_info()
    n_workers = info.num_cores * info.num_subcores
    assert n_edges % CH == 0
    g4 = _make_sc_gather_add(n_nodes, n_edges, d_out, n_workers)(p, q, ei3)
    gt = g4.transpose(0, 2, 1, 3).reshape(d_out, n_edges)

    blk = 32000
    grid = n_edges // blk
    out_t = pl.pallas_call(
        _out_body,
        grid=(grid,),
        in_specs=[
            pl.BlockSpec((d_edge, blk), lambda i: (0, i)),
            pl.BlockSpec((d_out, blk), lambda i: (0, i)),
            pl.BlockSpec((d_out, d_edge), lambda i: (0, 0)),
            pl.BlockSpec((d_out, 1), lambda i: (0, 0)),
        ],
        out_specs=pl.BlockSpec((d_out, blk), lambda i: (0, i)),
        out_shape=jax.ShapeDtypeStruct((d_out, n_edges), jnp.float32),
    )(ea_t, gt, wet, bcol)
    return out_t.T


# speedup vs baseline: 1.5869x; 1.1232x over previous
import functools

import jax
import jax.numpy as jnp
from jax import lax
from jax.experimental import pallas as pl
from jax.experimental.pallas import tpu as pltpu
from jax.experimental.pallas import tpu_sc as plsc

SUB = 80
NBUF = 5


def _pq_body(x8_ref, wsb_ref, wrb_ref, p_ref, q_ref):
    x8 = x8_ref[...]
    p_ref[...] = jnp.dot(x8, wsb_ref[...], preferred_element_type=jnp.float32)
    q_ref[...] = jnp.dot(x8, wrb_ref[...], preferred_element_type=jnp.float32)


def _out_body(ea_ref, g_ref, wet_ref, bcol_ref, o_ref):
    acc = jnp.dot(wet_ref[...], ea_ref[...], preferred_element_type=jnp.float32)
    o_ref[...] = jnp.maximum(acc + g_ref[...] + bcol_ref[...], 0.0)


def _make_sc_gather_add(n_nodes, n_edges, d_out, n_workers):
    mesh = plsc.VectorSubcoreMesh(core_axis_name="c", subcore_axis_name="s")
    epw = n_edges // n_workers
    cpw = epw // SUB
    outer = cpw // NBUF
    grp = NBUF * SUB

    scratch_types = [
        pltpu.VMEM((epw,), jnp.int32),
        pltpu.VMEM((epw,), jnp.int32),
        pltpu.VMEM((NBUF, SUB, d_out), jnp.float32),
        pltpu.VMEM((NBUF, SUB, d_out), jnp.float32),
        pltpu.VMEM((d_out * grp,), jnp.float32),
    ] + [pltpu.SemaphoreType.DMA] * (NBUF + 1)

    @functools.partial(
        pl.kernel,
        out_type=jax.ShapeDtypeStruct((d_out, n_edges), jnp.float32),
        mesh=mesh,
        scratch_types=scratch_types,
        compiler_params=pltpu.CompilerParams(
            use_tc_tiling_on_sc=False, needs_layout_passes=False),
    )
    def sc_kernel(p_hbm, q_hbm, ei2_hbm, gt_hbm,
                  sidx, didx, prow, qrow, gbuf, *sems):
        semg = sems[:NBUF]
        semo = sems[NBUF]
        wid = lax.axis_index("s") * 2 + lax.axis_index("c")
        pltpu.sync_copy(ei2_hbm.at[0, pl.ds(wid * epw, epw)], sidx)
        pltpu.sync_copy(ei2_hbm.at[1, pl.ds(wid * epw, epw)], didx)

        def fire(t, b):
            pltpu.make_async_copy(
                p_hbm.at[sidx.at[pl.ds(t * SUB, SUB)]], prow.at[b], semg[b]
            ).start()
            pltpu.make_async_copy(
                q_hbm.at[didx.at[pl.ds(t * SUB, SUB)]], qrow.at[b], semg[b]
            ).start()

        def wait_gathers(t, b):
            pltpu.make_async_copy(
                p_hbm.at[sidx.at[pl.ds(t * SUB, SUB)]], prow.at[b], semg[b]
            ).wait()
            pltpu.make_async_copy(
                q_hbm.at[didx.at[pl.ds(t * SUB, SUB)]], qrow.at[b], semg[b]
            ).wait()

        def out_copies(go):
            col0 = wid * epw + go * grp
            return [
                pltpu.make_async_copy(
                    gbuf.at[pl.ds(j * grp, grp)],
                    gt_hbm.at[j, pl.ds(col0, grp)],
                    semo,
                )
                for j in range(d_out)
            ]

        for b in range(NBUF):
            fire(b, b)

        ivec = lax.broadcasted_iota(jnp.int32, (16,), 0) * grp

        def step(go, carry):
            @pl.when(go > 0)
            def _():
                for c in out_copies(go - 1):
                    c.wait()

            for b in range(NBUF):
                t = go * NBUF + b
                wait_gathers(t, b)
                pb = prow.at[b]
                qb = qrow.at[b]
                base = b * SUB

                @plsc.parallel_loop(0, SUB, step=1, unroll=16)
                def _(i):
                    row = pb[i, :] + qb[i, :]
                    plsc.store_scatter(gbuf, [ivec + (base + i)], row)

                @pl.when(go < outer - 1)
                def _():
                    fire(t + NBUF, b)

            for c in out_copies(go):
                c.start()
            return carry

        lax.fori_loop(0, outer, step, 0)
        for c in out_copies(outer - 1):
            c.wait()

    return sc_kernel


def kernel(x, edge_index, edge_attr, W, b):
    n_nodes, d_in = x.shape
    n_edges, d_edge = edge_attr.shape
    d_out = W.shape[1]

    w_s = W[:d_in]
    w_r = W[d_in:2 * d_in]
    w_e = W[2 * d_in:]
    ei2 = edge_index.T
    ea_t = edge_attr.T
    wet = w_e.T
    bcol = b.reshape(d_out, 1)

    pack = 128 // d_out
    x8 = x.reshape(n_nodes // pack, pack * d_in)
    eye = jnp.eye(pack, dtype=jnp.float32)
    wsb = jnp.kron(eye, w_s)
    wrb = jnp.kron(eye, w_r)
    p128, q128 = pl.pallas_call(
        _pq_body,
        out_shape=(
            jax.ShapeDtypeStruct((n_nodes // pack, 128), jnp.float32),
            jax.ShapeDtypeStruct((n_nodes // pack, 128), jnp.float32),
        ),
    )(x8, wsb, wrb)
    p = p128.reshape(n_nodes, d_out)
    q = q128.reshape(n_nodes, d_out)

    info = plsc.get_sparse_core_info()
    n_workers = info.num_cores * info.num_subcores
    assert n_edges % (n_workers * SUB * NBUF) == 0
    gt = _make_sc_gather_add(n_nodes, n_edges, d_out, n_workers)(p, q, ei2)

    blk = 32000
    grid = n_edges // blk
    out_t = pl.pallas_call(
        _out_body,
        grid=(grid,),
        in_specs=[
            pl.BlockSpec((d_edge, blk), lambda i: (0, i)),
            pl.BlockSpec((d_out, blk), lambda i: (0, i)),
            pl.BlockSpec((d_out, d_edge), lambda i: (0, 0)),
            pl.BlockSpec((d_out, 1), lambda i: (0, 0)),
        ],
        out_specs=pl.BlockSpec((d_out, blk), lambda i: (0, i)),
        out_shape=jax.ShapeDtypeStruct((d_out, n_edges), jnp.float32),
    )(ea_t, gt, wet, bcol)
    return out_t.T


# speedup vs baseline: 1.6421x; 1.0348x over previous
import functools

import jax
import jax.numpy as jnp
from jax import lax
from jax.experimental import pallas as pl
from jax.experimental.pallas import tpu as pltpu
from jax.experimental.pallas import tpu_sc as plsc

SUB = 80
NBUF = 5


def _pq_body(x8_ref, wsb_ref, wrb_ref, p_ref, q_ref):
    x8 = x8_ref[...]
    p_ref[...] = jnp.dot(x8, wsb_ref[...], preferred_element_type=jnp.float32)
    q_ref[...] = jnp.dot(x8, wrb_ref[...], preferred_element_type=jnp.float32)


def _out_body(ea_ref, g_ref, wet_ref, bcol_ref, o_ref):
    acc = jnp.dot(wet_ref[...], ea_ref[...], preferred_element_type=jnp.float32)
    blkc = o_ref.shape[1] // 128
    i = pl.program_id(0)
    g = g_ref[:, pl.ds(i * blkc, blkc), :].reshape(o_ref.shape)
    o_ref[...] = jnp.maximum(acc + g + bcol_ref[...], 0.0)


def _make_sc_gather_add(n_nodes, n_edges, d_out, n_workers):
    mesh = plsc.VectorSubcoreMesh(core_axis_name="c", subcore_axis_name="s")
    epw = n_edges // n_workers
    cpw = epw // SUB
    outer = cpw // NBUF
    grp = NBUF * SUB

    scratch_types = [
        pltpu.VMEM((epw,), jnp.int32),
        pltpu.VMEM((epw,), jnp.int32),
        pltpu.VMEM((NBUF, SUB, d_out), jnp.float32),
        pltpu.VMEM((NBUF, SUB, d_out), jnp.float32),
        pltpu.VMEM((d_out * grp,), jnp.float32),
    ] + [pltpu.SemaphoreType.DMA] * (NBUF + 1)

    @functools.partial(
        pl.kernel,
        out_type=jax.ShapeDtypeStruct((d_out, n_edges), jnp.float32),
        mesh=mesh,
        scratch_types=scratch_types,
        compiler_params=pltpu.CompilerParams(
            use_tc_tiling_on_sc=False, needs_layout_passes=False),
    )
    def sc_kernel(p_hbm, q_hbm, ei2_hbm, gt_hbm,
                  sidx, didx, prow, qrow, gbuf, *sems):
        semg = sems[:NBUF]
        semo = sems[NBUF]
        wid = lax.axis_index("s") * 2 + lax.axis_index("c")
        pltpu.sync_copy(ei2_hbm.at[0, pl.ds(wid * epw, epw)], sidx)
        pltpu.sync_copy(ei2_hbm.at[1, pl.ds(wid * epw, epw)], didx)

        def fire(t, b):
            pltpu.make_async_copy(
                p_hbm.at[sidx.at[pl.ds(t * SUB, SUB)]], prow.at[b], semg[b]
            ).start()
            pltpu.make_async_copy(
                q_hbm.at[didx.at[pl.ds(t * SUB, SUB)]], qrow.at[b], semg[b]
            ).start()

        def wait_gathers(t, b):
            pltpu.make_async_copy(
                p_hbm.at[sidx.at[pl.ds(t * SUB, SUB)]], prow.at[b], semg[b]
            ).wait()
            pltpu.make_async_copy(
                q_hbm.at[didx.at[pl.ds(t * SUB, SUB)]], qrow.at[b], semg[b]
            ).wait()

        def out_copies(go):
            col0 = wid * epw + go * grp
            return [
                pltpu.make_async_copy(
                    gbuf.at[pl.ds(j * grp, grp)],
                    gt_hbm.at[j, pl.ds(col0, grp)],
                    semo,
                )
                for j in range(d_out)
            ]

        for b in range(NBUF):
            fire(b, b)

        ivec = lax.broadcasted_iota(jnp.int32, (16,), 0) * grp

        def step(go, carry):
            @pl.when(go > 0)
            def _():
                for c in out_copies(go - 1):
                    c.wait()

            for b in range(NBUF):
                t = go * NBUF + b
                wait_gathers(t, b)
                pb = prow.at[b]
                qb = qrow.at[b]
                base = b * SUB

                @plsc.parallel_loop(0, SUB, step=1, unroll=16)
                def _(i):
                    row = pb[i, :] + qb[i, :]
                    plsc.store_scatter(gbuf, [ivec + (base + i)], row)

                @pl.when(go < outer - 1)
                def _():
                    fire(t + NBUF, b)

            for c in out_copies(go):
                c.start()
            return carry

        lax.fori_loop(0, outer, step, 0)
        for c in out_copies(outer - 1):
            c.wait()

    return sc_kernel


def kernel(x, edge_index, edge_attr, W, b):
    n_nodes, d_in = x.shape
    n_edges, d_edge = edge_attr.shape
    d_out = W.shape[1]

    w_s = W[:d_in]
    w_r = W[d_in:2 * d_in]
    w_e = W[2 * d_in:]
    ei2 = edge_index.T
    ea_t = edge_attr.T
    wet = w_e.T
    bcol = b.reshape(d_out, 1)

    pack = 128 // d_out
    x8 = x.reshape(n_nodes // pack, pack * d_in)
    eye = jnp.eye(pack, dtype=jnp.float32)
    wsb = jnp.kron(eye, w_s)
    wrb = jnp.kron(eye, w_r)
    p128, q128 = pl.pallas_call(
        _pq_body,
        out_shape=(
            jax.ShapeDtypeStruct((n_nodes // pack, 128), jnp.float32),
            jax.ShapeDtypeStruct((n_nodes // pack, 128), jnp.float32),
        ),
    )(x8, wsb, wrb)
    p = p128.reshape(n_nodes, d_out)
    q = q128.reshape(n_nodes, d_out)

    info = plsc.get_sparse_core_info()
    n_workers = info.num_cores * info.num_subcores
    assert n_edges % (n_workers * SUB * NBUF) == 0
    gt = _make_sc_gather_add(n_nodes, n_edges, d_out, n_workers)(p, q, ei2)

    gt3 = gt.reshape(d_out, n_edges // 128, 128)
    blk = 32000
    grid = n_edges // blk
    out_t = pl.pallas_call(
        _out_body,
        grid=(grid,),
        in_specs=[
            pl.BlockSpec((d_edge, blk), lambda i: (0, i)),
            pl.BlockSpec((d_out, n_edges // 128, 128), lambda i: (0, 0, 0)),
            pl.BlockSpec((d_out, d_edge), lambda i: (0, 0)),
            pl.BlockSpec((d_out, 1), lambda i: (0, 0)),
        ],
        out_specs=pl.BlockSpec((d_out, blk), lambda i: (0, i)),
        out_shape=jax.ShapeDtypeStruct((d_out, n_edges), jnp.float32),
    )(ea_t, gt3, wet, bcol)
    return out_t.T
